# baseline (speedup 1.0000x reference)
import jax
import jax.numpy as jnp
from jax.experimental import pallas as pl
from jax.experimental.pallas import tpu as pltpu

BM = 128
LG = 128


def _gcn_kernel(d_ref, x_ref, a_ref, c_ref, w1t_ref, b1_ref,
                w2t_ref, b2_ref, w3t_ref, b3_ref, out_ref,
                mask_scr, xs_scr, acc1_scr, hh_scr, aux_scr):
    p = pl.program_id(0)
    t = pl.program_id(1)
    n = mask_scr.shape[1]
    bm = a_ref.shape[0]
    ns = n // bm
    f32 = jnp.float32
    bf16 = jnp.bfloat16

    @pl.when(jnp.logical_and(p == 0, t == 0))
    def _zero_xs():
        xs_scr[...] = jnp.zeros_like(xs_scr)

    @pl.when(p == 0)
    def _stream():
        deg = jnp.max(d_ref[...], axis=1, keepdims=True)
        rs = jax.lax.rsqrt(deg)
        aux_scr[pl.ds(t * bm, bm), 0:1] = rs
        xs = (x_ref[...] * rs).astype(bf16)
        xs_scr[pl.ds(t * bm, bm), :] = xs

        a = a_ref[...]
        mb = a.astype(bf16)
        mask_scr[pl.ds(t * bm, bm), :] = mb
        val = a * c_ref[...]
        idx = n - jnp.max(val, axis=1, keepdims=True)
        aux_scr[pl.ds(t * bm, bm), 1:2] = idx.astype(jnp.float32)

        acc1_scr[pl.ds(t * bm, bm), :] = jnp.dot(
            mb, xs_scr[...], preferred_element_type=f32)
        for i in range(32):
            if i < ns:
                @pl.when(i < t)
                def _backfill(i=i):
                    mbi = mask_scr[i * bm:(i + 1) * bm, pl.ds(t * bm, bm)]
                    acc1_scr[i * bm:(i + 1) * bm, :] += jnp.dot(
                        mbi, xs, preferred_element_type=f32)

    @pl.when(jnp.logical_and(p == 1, t == 0))
    def _finalize_layer1():
        idx = aux_scr[:, 1:2].astype(jnp.int32)
        q = idx // LG
        r = idx - q * LG
        ohq = (jax.lax.broadcasted_iota(jnp.int32, (n, n // LG), 1) == q
               ).astype(f32)
        ohr = (jax.lax.broadcasted_iota(jnp.int32, (n, LG), 1) == r
               ).astype(f32)
        rs2 = jnp.reshape(aux_scr[:, 0:1], (n // LG, LG))
        rsq = jnp.dot(ohq, rs2, preferred_element_type=f32)
        rsdj0 = jnp.sum(rsq * ohr, axis=1, keepdims=True)
        agg = acc1_scr[...] * rsdj0
        h = jnp.dot(agg, w1t_ref[...], preferred_element_type=f32)
        h = h + b1_ref[...]
        h = jnp.where(h > 0, h, 0.01 * h)
        hh_scr[...] = (h * aux_scr[:, 0:1]).astype(bf16)
        aux_scr[:, 2:3] = rsdj0

    @pl.when(p == 1)
    def _layer2():
        mb = mask_scr[pl.ds(t * bm, bm), :]
        acc = jnp.dot(mb, hh_scr[...], preferred_element_type=f32)
        agg = acc * aux_scr[pl.ds(t * bm, bm), 2:3]
        h = jnp.dot(agg, w2t_ref[...], preferred_element_type=f32)
        h = h + b2_ref[...]
        h = jnp.where(h > 0, h, 0.01 * h)
        o = jnp.dot(h, w3t_ref[...], preferred_element_type=f32)
        o = o + b3_ref[...]
        m = jnp.max(o, axis=1, keepdims=True)
        e = jnp.exp(o - m)
        out_ref[...] = (o - m) - jnp.log(jnp.sum(e, axis=1, keepdims=True))


def kernel(D, X, A, W1, b1, W2, b2, W3, b3):
    n, f = X.shape
    h1 = W1.shape[0]
    h2 = W2.shape[0]
    c = W3.shape[0]
    f32 = jnp.float32
    bf16 = jnp.bfloat16

    cvec = (n - jnp.arange(n, dtype=jnp.int32)).reshape(1, n)
    w1t = jnp.transpose(W1)
    w2t = jnp.transpose(W2)
    w3t = jnp.transpose(W3)
    b1r = jnp.reshape(b1, (1, h1))
    b2r = jnp.reshape(b2, (1, h2))
    b3r = jnp.reshape(b3, (1, c))

    ns = n // BM
    strip = lambda p, i: (jnp.where(p == 0, i, ns - 1), 0)
    out = pl.pallas_call(
        _gcn_kernel,
        grid=(2, ns),
        in_specs=[
            pl.BlockSpec((BM, n), strip),
            pl.BlockSpec((BM, f), strip),
            pl.BlockSpec((BM, n), strip),
            pl.BlockSpec((1, n), lambda p, i: (0, 0)),
            pl.BlockSpec((f, h1), lambda p, i: (0, 0)),
            pl.BlockSpec((1, h1), lambda p, i: (0, 0)),
            pl.BlockSpec((h1, h2), lambda p, i: (0, 0)),
            pl.BlockSpec((1, h2), lambda p, i: (0, 0)),
            pl.BlockSpec((h2, c), lambda p, i: (0, 0)),
            pl.BlockSpec((1, c), lambda p, i: (0, 0)),
        ],
        out_specs=pl.BlockSpec((BM, c), lambda p, i: (i, 0)),
        out_shape=jax.ShapeDtypeStruct((n, c), f32),
        scratch_shapes=[
            pltpu.VMEM((n, n), bf16),
            pltpu.VMEM((n, f), bf16),
            pltpu.VMEM((n, f), f32),
            pltpu.VMEM((n, h1), bf16),
            pltpu.VMEM((n, LG), f32),
        ],
    )(D, X, A, cvec, w1t, b1r, w2t, b2r, w3t, b3r)

    return out


# speedup vs baseline: 1.6760x; 1.6760x over previous
import jax
import jax.numpy as jnp
from jax.experimental import pallas as pl
from jax.experimental.pallas import tpu as pltpu

BM = 256
LG = 128


def _gcn_kernel(d_ref, x_ref, a_ref, c_ref, w1t_ref, b1_ref,
                w2t_ref, b2_ref, w3t_ref, b3_ref, out_ref,
                mask_scr, xs_scr, hh_scr, aux_scr, rs2_scr):
    p = pl.program_id(0)
    t = pl.program_id(1)
    n = mask_scr.shape[1]
    bm = a_ref.shape[0]
    f32 = jnp.float32
    bf16 = jnp.bfloat16

    @pl.when(p == 0)
    def _stream():
        deg = jnp.max(d_ref[...], axis=1, keepdims=True)
        rs = jax.lax.rsqrt(deg)
        aux_scr[pl.ds(t * bm, bm), 0:1] = rs
        xs_scr[pl.ds(t * bm, bm), :] = (x_ref[...] * rs).astype(bf16)

        a = a_ref[...]
        mask_scr[pl.ds(t * bm, bm), :] = a.astype(bf16)
        val = a * c_ref[...]
        idx = n - jnp.max(val, axis=1, keepdims=True)
        aux_scr[pl.ds(t * bm, bm), 1:2] = idx.astype(f32)

    @pl.when(jnp.logical_and(p == 1, t == 0))
    def _pack_rs2():
        rs2_scr[...] = jnp.reshape(aux_scr[:, 0:1], (n // LG, LG))

    @pl.when(p == 1)
    def _layer1():
        sl = pl.ds(t * bm, bm)
        idx = aux_scr[sl, 1:2].astype(jnp.int32)
        q = idx // LG
        r = idx - q * LG
        ohq = (jax.lax.broadcasted_iota(jnp.int32, (bm, n // LG), 1) == q
               ).astype(f32)
        ohr = (jax.lax.broadcasted_iota(jnp.int32, (bm, LG), 1) == r
               ).astype(f32)
        rsq = jnp.dot(ohq, rs2_scr[...], preferred_element_type=f32)
        rsdj0 = jnp.sum(rsq * ohr, axis=1, keepdims=True)
        aux_scr[sl, 2:3] = rsdj0

        mb = mask_scr[sl, :]
        acc = jnp.dot(mb, xs_scr[...], preferred_element_type=f32)
        agg = acc * rsdj0
        h = jnp.dot(agg, w1t_ref[...], preferred_element_type=f32)
        h = h + b1_ref[...]
        h = jnp.where(h > 0, h, 0.01 * h)
        hh_scr[sl, :] = (h * aux_scr[sl, 0:1]).astype(bf16)

    @pl.when(p == 2)
    def _layer2():
        sl = pl.ds(t * bm, bm)
        mb = mask_scr[sl, :]
        acc = jnp.dot(mb, hh_scr[...], preferred_element_type=f32)
        agg = acc * aux_scr[sl, 2:3]
        h = jnp.dot(agg, w2t_ref[...], preferred_element_type=f32)
        h = h + b2_ref[...]
        h = jnp.where(h > 0, h, 0.01 * h)
        o = jnp.dot(h, w3t_ref[...], preferred_element_type=f32)
        o = o + b3_ref[...]
        m = jnp.max(o, axis=1, keepdims=True)
        e = jnp.exp(o - m)
        out_ref[...] = (o - m) - jnp.log(jnp.sum(e, axis=1, keepdims=True))


def kernel(D, X, A, W1, b1, W2, b2, W3, b3):
    n, f = X.shape
    h1 = W1.shape[0]
    h2 = W2.shape[0]
    c = W3.shape[0]
    f32 = jnp.float32
    bf16 = jnp.bfloat16

    cvec = (n - jnp.arange(n, dtype=jnp.int32)).reshape(1, n)
    w1t = jnp.transpose(W1)
    w2t = jnp.transpose(W2)
    w3t = jnp.transpose(W3)
    b1r = jnp.reshape(b1, (1, h1))
    b2r = jnp.reshape(b2, (1, h2))
    b3r = jnp.reshape(b3, (1, c))

    ns = n // BM
    strip = lambda p, i: (jnp.where(p == 0, i, ns - 1), 0)
    out = pl.pallas_call(
        _gcn_kernel,
        grid=(3, ns),
        in_specs=[
            pl.BlockSpec((BM, n), strip),
            pl.BlockSpec((BM, f), strip),
            pl.BlockSpec((BM, n), strip),
            pl.BlockSpec((1, n), lambda p, i: (0, 0)),
            pl.BlockSpec((f, h1), lambda p, i: (0, 0)),
            pl.BlockSpec((1, h1), lambda p, i: (0, 0)),
            pl.BlockSpec((h1, h2), lambda p, i: (0, 0)),
            pl.BlockSpec((1, h2), lambda p, i: (0, 0)),
            pl.BlockSpec((h2, c), lambda p, i: (0, 0)),
            pl.BlockSpec((1, c), lambda p, i: (0, 0)),
        ],
        out_specs=pl.BlockSpec((BM, c), lambda p, i: (i, 0)),
        out_shape=jax.ShapeDtypeStruct((n, c), f32),
        scratch_shapes=[
            pltpu.VMEM((n, n), bf16),
            pltpu.VMEM((n, f), bf16),
            pltpu.VMEM((n, h1), bf16),
            pltpu.VMEM((n, LG), f32),
            pltpu.VMEM((n // LG, LG), f32),
        ],
    )(D, X, A, cvec, w1t, b1r, w2t, b2r, w3t, b3r)

    return out


# speedup vs baseline: 1.8722x; 1.1171x over previous
import jax
import jax.numpy as jnp
from jax.experimental import pallas as pl
from jax.experimental.pallas import tpu as pltpu

BM = 512
BR = 1024
LG = 128


def _split(v):
    hi = v.astype(jnp.bfloat16)
    lo = (v - hi.astype(jnp.float32)).astype(jnp.bfloat16)
    return hi, lo


def _prep_kernel(d_ref, x_ref, xh_ref, rs_ref):
    deg = jnp.max(d_ref[...], axis=1, keepdims=True)
    rs = jax.lax.rsqrt(deg)
    xh_ref[...] = (x_ref[...] * rs).astype(jnp.bfloat16)
    rs_ref[...] = rs


def _gcn_kernel(a_ref, c_ref, rs_ref, rs2_ref, xh_ref, w1t_ref, b1_ref,
                w2t_ref, b2_ref, w3t_ref, b3_ref, out_ref,
                mask_scr, hh_scr, rsdj0_scr):
    p = pl.program_id(0)
    i = pl.program_id(1)
    n = mask_scr.shape[1]
    bm = a_ref.shape[0]

    @pl.when(p == 0)
    def _layer1():
        a = a_ref[...]
        mb = a.astype(jnp.bfloat16)
        mask_scr[pl.ds(i * bm, bm), :] = mb
        val = a * c_ref[...]
        idx = n - jnp.max(val, axis=1, keepdims=True)
        q = idx // LG
        r = idx - q * LG
        ohq = (jax.lax.broadcasted_iota(jnp.int32, (bm, n // LG), 1) == q
               ).astype(jnp.float32)
        ohr = (jax.lax.broadcasted_iota(jnp.int32, (bm, LG), 1) == r
               ).astype(jnp.float32)
        rsq = jnp.dot(ohq, rs2_ref[...], preferred_element_type=jnp.float32)
        rsdj0 = jnp.sum(rsq * ohr, axis=1, keepdims=True)
        rsdj0_scr[pl.ds(i * bm, bm), :] = rsdj0

        acc = jnp.dot(mb, xh_ref[...], preferred_element_type=jnp.float32)
        agg = acc * rsdj0
        h = jnp.dot(agg, w1t_ref[...], preferred_element_type=jnp.float32)
        h = h + b1_ref[...]
        h = jnp.where(h > 0, h, 0.01 * h)
        rsi = rs_ref[pl.ds(i * bm, bm), :]
        hh_scr[pl.ds(i * bm, bm), :] = (h * rsi).astype(jnp.bfloat16)

    @pl.when(p == 1)
    def _layer2():
        mb = mask_scr[pl.ds(i * bm, bm), :]
        acc = jnp.dot(mb, hh_scr[...], preferred_element_type=jnp.float32)
        agg = acc * rsdj0_scr[pl.ds(i * bm, bm), :]
        h = jnp.dot(agg, w2t_ref[...], preferred_element_type=jnp.float32)
        h = h + b2_ref[...]
        h = jnp.where(h > 0, h, 0.01 * h)
        o = jnp.dot(h, w3t_ref[...], preferred_element_type=jnp.float32)
        o = o + b3_ref[...]
        m = jnp.max(o, axis=1, keepdims=True)
        e = jnp.exp(o - m)
        out_ref[...] = (o - m) - jnp.log(jnp.sum(e, axis=1, keepdims=True))


def kernel(D, X, A, W1, b1, W2, b2, W3, b3):
    n, f = X.shape
    h1 = W1.shape[0]
    h2 = W2.shape[0]
    c = W3.shape[0]
    f32 = jnp.float32
    bf16 = jnp.bfloat16

    xh, rsdeg = pl.pallas_call(
        _prep_kernel,
        grid=(n // BR,),
        in_specs=[
            pl.BlockSpec((BR, n), lambda i: (i, 0)),
            pl.BlockSpec((BR, f), lambda i: (i, 0)),
        ],
        out_specs=[
            pl.BlockSpec((BR, f), lambda i: (i, 0)),
            pl.BlockSpec((BR, 1), lambda i: (i, 0)),
        ],
        out_shape=[
            jax.ShapeDtypeStruct((n, f), bf16),
            jax.ShapeDtypeStruct((n, 1), f32),
        ],
    )(D, X)

    cvec = (n - jnp.arange(n, dtype=jnp.int32)).reshape(1, n)
    rs2 = jnp.reshape(rsdeg, (n // LG, LG))

    w1t = jnp.transpose(W1)
    w2t = jnp.transpose(W2)
    w3t = jnp.transpose(W3)
    b1r = jnp.reshape(b1, (1, h1))
    b2r = jnp.reshape(b2, (1, h2))
    b3r = jnp.reshape(b3, (1, c))

    ns = n // BM
    out = pl.pallas_call(
        _gcn_kernel,
        grid=(2, ns),
        in_specs=[
            pl.BlockSpec((BM, n), lambda p, i: (jnp.where(p == 0, i, ns - 1), 0)),
            pl.BlockSpec((1, n), lambda p, i: (0, 0)),
            pl.BlockSpec((n, 1), lambda p, i: (0, 0)),
            pl.BlockSpec((n // LG, LG), lambda p, i: (0, 0)),
            pl.BlockSpec((n, f), lambda p, i: (0, 0)),
            pl.BlockSpec((f, h1), lambda p, i: (0, 0)),
            pl.BlockSpec((1, h1), lambda p, i: (0, 0)),
            pl.BlockSpec((h1, h2), lambda p, i: (0, 0)),
            pl.BlockSpec((1, h2), lambda p, i: (0, 0)),
            pl.BlockSpec((h2, c), lambda p, i: (0, 0)),
            pl.BlockSpec((1, c), lambda p, i: (0, 0)),
        ],
        out_specs=pl.BlockSpec((BM, c), lambda p, i: (i, 0)),
        out_shape=jax.ShapeDtypeStruct((n, c), f32),
        scratch_shapes=[
            pltpu.VMEM((n, n), bf16),
            pltpu.VMEM((n, h1), bf16),
            pltpu.VMEM((n, 1), f32),
        ],
    )(A, cvec, rsdeg, rs2, xh, w1t, b1r, w2t, b2r, w3t, b3r)

    return out


# speedup vs baseline: 1.8835x; 1.0061x over previous
import jax
import jax.numpy as jnp
from jax.experimental import pallas as pl
from jax.experimental.pallas import tpu as pltpu

BM = 512
BR = 512
LG = 128


def _prep_kernel(d_ref, x_ref, xh_ref, rs_ref):
    deg = jnp.max(d_ref[...], axis=1, keepdims=True)
    rs = jax.lax.rsqrt(deg)
    xh_ref[...] = (x_ref[...] * rs).astype(jnp.bfloat16)
    rs_ref[...] = rs


def _gcn_kernel(a_ref, c_ref, rs_ref, rs2_ref, xh_ref, w1t_ref, b1_ref,
                w2t_ref, b2_ref, w3t_ref, b3_ref, out_ref,
                mask_scr, hh_scr, rsdj0_scr):
    p = pl.program_id(0)
    i = pl.program_id(1)
    n = mask_scr.shape[1]
    bm = a_ref.shape[0]

    @pl.when(p == 0)
    def _layer1():
        a = a_ref[...]
        mb = a.astype(jnp.bfloat16)
        mask_scr[pl.ds(i * bm, bm), :] = mb
        val = a * c_ref[...]
        idx = n - jnp.max(val, axis=1, keepdims=True)
        q = idx // LG
        r = idx - q * LG
        ohq = (jax.lax.broadcasted_iota(jnp.int32, (bm, n // LG), 1) == q
               ).astype(jnp.float32)
        ohr = (jax.lax.broadcasted_iota(jnp.int32, (bm, LG), 1) == r
               ).astype(jnp.float32)
        rsq = jnp.dot(ohq, rs2_ref[...], preferred_element_type=jnp.float32)
        rsdj0 = jnp.sum(rsq * ohr, axis=1, keepdims=True)
        rsdj0_scr[pl.ds(i * bm, bm), :] = rsdj0

        acc = jnp.dot(mb, xh_ref[...], preferred_element_type=jnp.float32)
        agg = acc * rsdj0
        h = jnp.dot(agg, w1t_ref[...], preferred_element_type=jnp.float32)
        h = h + b1_ref[...]
        h = jnp.where(h > 0, h, 0.01 * h)
        rsi = rs_ref[pl.ds(i * bm, bm), :]
        hh_scr[pl.ds(i * bm, bm), :] = (h * rsi).astype(jnp.bfloat16)

    @pl.when(p == 1)
    def _layer2():
        mb = mask_scr[pl.ds(i * bm, bm), :]
        acc = jnp.dot(mb, hh_scr[...], preferred_element_type=jnp.float32)
        agg = acc * rsdj0_scr[pl.ds(i * bm, bm), :]
        h = jnp.dot(agg, w2t_ref[...], preferred_element_type=jnp.float32)
        h = h + b2_ref[...]
        h = jnp.where(h > 0, h, 0.01 * h)
        o = jnp.dot(h, w3t_ref[...], preferred_element_type=jnp.float32)
        o = o + b3_ref[...]
        m = jnp.max(o, axis=1, keepdims=True)
        e = jnp.exp(o - m)
        out_ref[...] = (o - m) - jnp.log(jnp.sum(e, axis=1, keepdims=True))


def kernel(D, X, A, W1, b1, W2, b2, W3, b3):
    n, f = X.shape
    h1 = W1.shape[0]
    h2 = W2.shape[0]
    c = W3.shape[0]
    f32 = jnp.float32
    bf16 = jnp.bfloat16

    xh, rsdeg = pl.pallas_call(
        _prep_kernel,
        grid=(n // BR,),
        in_specs=[
            pl.BlockSpec((BR, n), lambda i: (i, 0)),
            pl.BlockSpec((BR, f), lambda i: (i, 0)),
        ],
        out_specs=[
            pl.BlockSpec((BR, f), lambda i: (i, 0)),
            pl.BlockSpec((BR, 1), lambda i: (i, 0)),
        ],
        out_shape=[
            jax.ShapeDtypeStruct((n, f), bf16),
            jax.ShapeDtypeStruct((n, 1), f32),
        ],
    )(D, X)

    cvec = (n - jnp.arange(n, dtype=jnp.int32)).reshape(1, n)
    rs2 = jnp.reshape(rsdeg, (n // LG, LG))

    w1t = jnp.transpose(W1)
    w2t = jnp.transpose(W2)
    w3t = jnp.transpose(W3)
    b1r = jnp.reshape(b1, (1, h1))
    b2r = jnp.reshape(b2, (1, h2))
    b3r = jnp.reshape(b3, (1, c))

    ns = n // BM
    out = pl.pallas_call(
        _gcn_kernel,
        grid=(2, ns),
        in_specs=[
            pl.BlockSpec((BM, n), lambda p, i: (jnp.where(p == 0, i, ns - 1), 0)),
            pl.BlockSpec((1, n), lambda p, i: (0, 0)),
            pl.BlockSpec((n, 1), lambda p, i: (0, 0)),
            pl.BlockSpec((n // LG, LG), lambda p, i: (0, 0)),
            pl.BlockSpec((n, f), lambda p, i: (0, 0)),
            pl.BlockSpec((f, h1), lambda p, i: (0, 0)),
            pl.BlockSpec((1, h1), lambda p, i: (0, 0)),
            pl.BlockSpec((h1, h2), lambda p, i: (0, 0)),
            pl.BlockSpec((1, h2), lambda p, i: (0, 0)),
            pl.BlockSpec((h2, c), lambda p, i: (0, 0)),
            pl.BlockSpec((1, c), lambda p, i: (0, 0)),
        ],
        out_specs=pl.BlockSpec((BM, c), lambda p, i: (i, 0)),
        out_shape=jax.ShapeDtypeStruct((n, c), f32),
        scratch_shapes=[
            pltpu.VMEM((n, n), bf16),
            pltpu.VMEM((n, h1), bf16),
            pltpu.VMEM((n, 1), f32),
        ],
    )(A, cvec, rsdeg, rs2, xh, w1t, b1r, w2t, b2r, w3t, b3r)

    return out
